# baseline (speedup 1.0000x reference)
import functools

import jax
import jax.numpy as jnp
from jax import lax
from jax.experimental import pallas as pl
from jax.experimental.pallas import tpu as pltpu
from jax.experimental.pallas import tpu_sc as plsc

DIM = 300
BATCH = 16384
VOCAB = 75000

_NC, _NS = 2, 16
NW = _NC * _NS
_G = 16


@functools.cache
def _sc_sweep_fn():
    mesh = plsc.VectorSubcoreMesh(core_axis_name="c", subcore_axis_name="s")

    @functools.partial(
        pl.kernel,
        mesh=mesh,
        compiler_params=pltpu.CompilerParams(needs_layout_passes=False),
        out_type=[
            jax.ShapeDtypeStruct((DIM, BATCH), jnp.float32),
            jax.ShapeDtypeStruct((DIM, 2 * BATCH), jnp.float32),
        ],
        scratch_types=[
            pltpu.VMEM((VOCAB,), jnp.float32),
            pltpu.VMEM((BATCH,), jnp.int32),
            pltpu.VMEM((BATCH,), jnp.float32),
            pltpu.SemaphoreType.DMA,
        ],
    )
    def _sc_sweep(src_t, tgt_t, src_idx, tgt_idx, src_out, full_out,
                  row_v, idx_v, out_v, sem):
        wid = lax.axis_index("s") * _NC + lax.axis_index("c")
        for tab, idx, write_out in (
                (src_t, src_idx, lambda d: src_out.at[d]),
                (tgt_t, tgt_idx, lambda d: full_out.at[d, pl.ds(BATCH,
                                                                BATCH)])):
            pltpu.sync_copy(idx, idx_v)

            @pl.loop(0, (DIM - 1) // NW + 1)
            def _(r):
                d = wid + r * NW

                @pl.when(d < DIM)
                def _():
                    pltpu.sync_copy(tab.at[d], row_v)

                    @pl.loop(0, BATCH // _G, unroll=4)
                    def _(g):
                        vec = idx_v[pl.ds(g * _G, _G)]
                        out_v[pl.ds(g * _G, _G)] = plsc.load_gather(
                            row_v, [vec])

                    pltpu.sync_copy(out_v, write_out(d))

    return _sc_sweep


BN = 2048


def _mm_t_body(src_ref, we_ref, wd_ref, b_ref, full_ref, ot_ref,
               m_ref, bv_ref):
    del full_ref
    @pl.when(pl.program_id(0) == 0)
    def _():
        m_ref[...] = lax.dot_general(
            we_ref[...], wd_ref[...],
            dimension_numbers=(((0,), (0,)), ((), ())),
            preferred_element_type=jnp.float32)
        bv_ref[...] = lax.dot_general(
            wd_ref[...], b_ref[...],
            dimension_numbers=(((0,), (1,)), ((), ())),
            preferred_element_type=jnp.float32)

    ot_ref[...] = lax.dot_general(
        m_ref[...], src_ref[...],
        dimension_numbers=(((0,), (0,)), ((), ())),
        preferred_element_type=jnp.float32) + bv_ref[...]


_mm_t = pl.pallas_call(
    _mm_t_body,
    grid=(BATCH // BN,),
    in_specs=[
        pl.BlockSpec((DIM, BN), lambda i: (0, i)),
        pl.BlockSpec((DIM, DIM), lambda i: (0, 0)),
        pl.BlockSpec((DIM, DIM), lambda i: (0, 0)),
        pl.BlockSpec((1, DIM), lambda i: (0, 0)),
        pl.BlockSpec(memory_space=pltpu.MemorySpace.HBM),
    ],
    out_specs=pl.BlockSpec((DIM, BN), lambda i: (0, i)),
    out_shape=jax.ShapeDtypeStruct((DIM, 2 * BATCH), jnp.float32),
    input_output_aliases={4: 0},
    scratch_shapes=[
        pltpu.VMEM((DIM, DIM), jnp.float32),
        pltpu.VMEM((DIM, 1), jnp.float32),
    ],
)


def kernel(emb_src, emb_tgt, W_enc, b_enc, W_dec, src_id, tgt_id):
    src_t, full_t = _sc_sweep_fn()(
        jnp.transpose(emb_src), jnp.transpose(emb_tgt),
        src_id.astype(jnp.int32), tgt_id.astype(jnp.int32))
    out_t = _mm_t(src_t, W_enc, W_dec, b_enc.reshape(1, DIM), full_t)
    return jnp.transpose(out_t)


# speedup vs baseline: 1.0132x; 1.0132x over previous
import functools

import jax
import jax.numpy as jnp
from jax import lax
from jax.experimental import pallas as pl
from jax.experimental.pallas import tpu as pltpu
from jax.experimental.pallas import tpu_sc as plsc

DIM = 300
BATCH = 16384
VOCAB = 75000

_NC, _NS = 2, 16
NW = _NC * _NS
_G = 16


@functools.cache
def _sc_sweep_fn():
    mesh = plsc.VectorSubcoreMesh(core_axis_name="c", subcore_axis_name="s")

    @functools.partial(
        pl.kernel,
        mesh=mesh,
        compiler_params=pltpu.CompilerParams(needs_layout_passes=False),
        out_type=[
            jax.ShapeDtypeStruct((DIM, BATCH), jnp.float32),
            jax.ShapeDtypeStruct((DIM, 2 * BATCH), jnp.float32),
        ],
        scratch_types=[
            pltpu.VMEM((VOCAB,), jnp.float32),
            pltpu.VMEM((BATCH,), jnp.int32),
            pltpu.VMEM((BATCH,), jnp.float32),
            pltpu.SemaphoreType.DMA,
        ],
    )
    def _sc_sweep(src_t, tgt_t, src_idx, tgt_idx, src_out, full_out,
                  row_v, idx_v, out_v, sem):
        wid = lax.axis_index("s") * _NC + lax.axis_index("c")
        for tab, idx, write_out in (
                (src_t, src_idx, lambda d: src_out.at[d]),
                (tgt_t, tgt_idx, lambda d: full_out.at[d, pl.ds(BATCH,
                                                                BATCH)])):
            pltpu.sync_copy(idx, idx_v)

            @pl.loop(0, (DIM - 1) // NW + 1)
            def _(r):
                d = wid + r * NW

                @pl.when(d < DIM)
                def _():
                    pltpu.sync_copy(tab.at[d], row_v)

                    @pl.loop(0, BATCH // _G, unroll=16)
                    def _(g):
                        vec = idx_v[pl.ds(g * _G, _G)]
                        out_v[pl.ds(g * _G, _G)] = plsc.load_gather(
                            row_v, [vec])

                    pltpu.sync_copy(out_v, write_out(d))

    return _sc_sweep


BN = 2048


def _mm_t_body(src_ref, we_ref, wd_ref, b_ref, full_ref, ot_ref,
               m_ref, bv_ref):
    del full_ref
    @pl.when(pl.program_id(0) == 0)
    def _():
        m_ref[...] = lax.dot_general(
            we_ref[...], wd_ref[...],
            dimension_numbers=(((0,), (0,)), ((), ())),
            preferred_element_type=jnp.float32)
        bv_ref[...] = lax.dot_general(
            wd_ref[...], b_ref[...],
            dimension_numbers=(((0,), (1,)), ((), ())),
            preferred_element_type=jnp.float32)

    ot_ref[...] = lax.dot_general(
        m_ref[...], src_ref[...],
        dimension_numbers=(((0,), (0,)), ((), ())),
        preferred_element_type=jnp.float32) + bv_ref[...]


_mm_t = pl.pallas_call(
    _mm_t_body,
    grid=(BATCH // BN,),
    in_specs=[
        pl.BlockSpec((DIM, BN), lambda i: (0, i)),
        pl.BlockSpec((DIM, DIM), lambda i: (0, 0)),
        pl.BlockSpec((DIM, DIM), lambda i: (0, 0)),
        pl.BlockSpec((1, DIM), lambda i: (0, 0)),
        pl.BlockSpec(memory_space=pltpu.MemorySpace.HBM),
    ],
    out_specs=pl.BlockSpec((DIM, BN), lambda i: (0, i)),
    out_shape=jax.ShapeDtypeStruct((DIM, 2 * BATCH), jnp.float32),
    input_output_aliases={4: 0},
    scratch_shapes=[
        pltpu.VMEM((DIM, DIM), jnp.float32),
        pltpu.VMEM((DIM, 1), jnp.float32),
    ],
)


def kernel(emb_src, emb_tgt, W_enc, b_enc, W_dec, src_id, tgt_id):
    src_t, full_t = _sc_sweep_fn()(
        jnp.transpose(emb_src), jnp.transpose(emb_tgt),
        src_id.astype(jnp.int32), tgt_id.astype(jnp.int32))
    out_t = _mm_t(src_t, W_enc, W_dec, b_enc.reshape(1, DIM), full_t)
    return jnp.transpose(out_t)


# speedup vs baseline: 1.9763x; 1.9506x over previous
import functools

import jax
import jax.numpy as jnp
from jax import lax
from jax.experimental import pallas as pl
from jax.experimental.pallas import tpu as pltpu
from jax.experimental.pallas import tpu_sc as plsc

DIM = 300
BATCH = 16384
VOCAB = 75000

_NC, _NS = 2, 16
NW = _NC * _NS
_G = 16


@functools.cache
def _sc_sweep_fn():
    mesh = plsc.VectorSubcoreMesh(core_axis_name="c", subcore_axis_name="s")

    @functools.partial(
        pl.kernel,
        mesh=mesh,
        compiler_params=pltpu.CompilerParams(needs_layout_passes=False),
        out_type=[
            jax.ShapeDtypeStruct((DIM, BATCH), jnp.float32),
            jax.ShapeDtypeStruct((DIM, 2 * BATCH), jnp.float32),
        ],
        scratch_types=[
            pltpu.VMEM((VOCAB,), jnp.float32),
            pltpu.VMEM((BATCH,), jnp.int32),
            pltpu.VMEM((BATCH,), jnp.float32),
            pltpu.SemaphoreType.DMA,
        ],
    )
    def _sc_sweep(src_t, tgt_t, src_idx, tgt_idx, src_out, full_out,
                  row_v, idx_v, out_v, sem):
        wid = lax.axis_index("s") * _NC + lax.axis_index("c")
        for tab, idx, write_out in (
                (src_t, src_idx, lambda d: src_out.at[d]),
                (tgt_t, tgt_idx, lambda d: full_out.at[d, pl.ds(BATCH,
                                                                BATCH)])):
            pltpu.sync_copy(idx, idx_v)

            @pl.loop(0, (DIM - 1) // NW + 1)
            def _(r):
                d = wid + r * NW

                @pl.when(d < DIM)
                def _():
                    pltpu.sync_copy(tab.at[d], row_v)

                    W = 8

                    @pl.loop(0, BATCH // (_G * W))
                    def _(g):
                        vecs = [idx_v[pl.ds((g * W + k) * _G, _G)]
                                for k in range(W)]
                        outs = [plsc.load_gather(row_v, [v])
                                for v in vecs]
                        for k in range(W):
                            out_v[pl.ds((g * W + k) * _G, _G)] = outs[k]

                    pltpu.sync_copy(out_v, write_out(d))

    return _sc_sweep


BN = 2048


def _mm_t_body(src_ref, we_ref, wd_ref, b_ref, full_ref, ot_ref,
               m_ref, bv_ref):
    del full_ref
    @pl.when(pl.program_id(0) == 0)
    def _():
        m_ref[...] = lax.dot_general(
            we_ref[...], wd_ref[...],
            dimension_numbers=(((0,), (0,)), ((), ())),
            preferred_element_type=jnp.float32)
        bv_ref[...] = lax.dot_general(
            wd_ref[...], b_ref[...],
            dimension_numbers=(((0,), (1,)), ((), ())),
            preferred_element_type=jnp.float32)

    ot_ref[...] = lax.dot_general(
        m_ref[...], src_ref[...],
        dimension_numbers=(((0,), (0,)), ((), ())),
        preferred_element_type=jnp.float32) + bv_ref[...]


_mm_t = pl.pallas_call(
    _mm_t_body,
    grid=(BATCH // BN,),
    in_specs=[
        pl.BlockSpec((DIM, BN), lambda i: (0, i)),
        pl.BlockSpec((DIM, DIM), lambda i: (0, 0)),
        pl.BlockSpec((DIM, DIM), lambda i: (0, 0)),
        pl.BlockSpec((1, DIM), lambda i: (0, 0)),
        pl.BlockSpec(memory_space=pltpu.MemorySpace.HBM),
    ],
    out_specs=pl.BlockSpec((DIM, BN), lambda i: (0, i)),
    out_shape=jax.ShapeDtypeStruct((DIM, 2 * BATCH), jnp.float32),
    input_output_aliases={4: 0},
    scratch_shapes=[
        pltpu.VMEM((DIM, DIM), jnp.float32),
        pltpu.VMEM((DIM, 1), jnp.float32),
    ],
)


def kernel(emb_src, emb_tgt, W_enc, b_enc, W_dec, src_id, tgt_id):
    src_t, full_t = _sc_sweep_fn()(
        jnp.transpose(emb_src), jnp.transpose(emb_tgt),
        src_id.astype(jnp.int32), tgt_id.astype(jnp.int32))
    out_t = _mm_t(src_t, W_enc, W_dec, b_enc.reshape(1, DIM), full_t)
    return jnp.transpose(out_t)


# speedup vs baseline: 2.1118x; 1.0685x over previous
import functools

import jax
import jax.numpy as jnp
from jax import lax
from jax.experimental import pallas as pl
from jax.experimental.pallas import tpu as pltpu
from jax.experimental.pallas import tpu_sc as plsc

DIM = 300
BATCH = 16384
VOCAB = 75000

_NC, _NS = 2, 16
NW = _NC * _NS
_G = 16


@functools.cache
def _sc_sweep_fn():
    mesh = plsc.VectorSubcoreMesh(core_axis_name="c", subcore_axis_name="s")

    @functools.partial(
        pl.kernel,
        mesh=mesh,
        compiler_params=pltpu.CompilerParams(needs_layout_passes=False),
        out_type=[
            jax.ShapeDtypeStruct((DIM, BATCH), jnp.float32),
            jax.ShapeDtypeStruct((DIM, 2 * BATCH), jnp.float32),
        ],
        scratch_types=[
            pltpu.VMEM((VOCAB,), jnp.float32),
            pltpu.VMEM((BATCH,), jnp.int32),
            pltpu.VMEM((BATCH,), jnp.float32),
            pltpu.VMEM((BATCH,), jnp.float32),
            pltpu.SemaphoreType.DMA,
            pltpu.SemaphoreType.DMA,
        ],
    )
    def _sc_sweep(src_t, tgt_t, src_idx, tgt_idx, src_out, full_out,
                  row_v, idx_v, out0, out1, sem0, sem1):
        wid = lax.axis_index("s") * _NC + lax.axis_index("c")
        out_bufs = (out0, out1)
        sems = (sem0, sem1)
        n_rows = (DIM - 1) // NW + 1
        for tab, idx, write_out in (
                (src_t, src_idx, lambda d: src_out.at[d]),
                (tgt_t, tgt_idx, lambda d: full_out.at[d, pl.ds(BATCH,
                                                                BATCH)])):
            pltpu.sync_copy(idx, idx_v)

            @pl.loop(0, n_rows, step=2)
            def _(r0):
                for half in range(2):
                    r = r0 + half
                    d = wid + r * NW

                    @pl.when(d < DIM)
                    def _():
                        pltpu.sync_copy(tab.at[d], row_v)

                        @pl.when(r >= 2)
                        def _():
                            pltpu.make_async_copy(
                                out_bufs[half], write_out(d),
                                sems[half]).wait()

                        W = 8

                        @pl.loop(0, BATCH // (_G * W))
                        def _(g):
                            vecs = [idx_v[pl.ds((g * W + k) * _G, _G)]
                                    for k in range(W)]
                            res = [plsc.load_gather(row_v, [v])
                                   for v in vecs]
                            for k in range(W):
                                out_bufs[half][
                                    pl.ds((g * W + k) * _G, _G)] = res[k]

                        pltpu.async_copy(out_bufs[half], write_out(d),
                                         sems[half])

            for half in range(2):
                pltpu.make_async_copy(out_bufs[half], write_out(0),
                                      sems[half]).wait()

    return _sc_sweep


BN = 2048


def _mm_t_body(src_ref, we_ref, wd_ref, b_ref, full_ref, ot_ref,
               m_ref, bv_ref):
    del full_ref
    @pl.when(pl.program_id(0) == 0)
    def _():
        m_ref[...] = lax.dot_general(
            we_ref[...], wd_ref[...],
            dimension_numbers=(((0,), (0,)), ((), ())),
            preferred_element_type=jnp.float32)
        bv_ref[...] = lax.dot_general(
            wd_ref[...], b_ref[...],
            dimension_numbers=(((0,), (1,)), ((), ())),
            preferred_element_type=jnp.float32)

    ot_ref[...] = lax.dot_general(
        m_ref[...], src_ref[...],
        dimension_numbers=(((0,), (0,)), ((), ())),
        preferred_element_type=jnp.float32) + bv_ref[...]


_mm_t = pl.pallas_call(
    _mm_t_body,
    grid=(BATCH // BN,),
    in_specs=[
        pl.BlockSpec((DIM, BN), lambda i: (0, i)),
        pl.BlockSpec((DIM, DIM), lambda i: (0, 0)),
        pl.BlockSpec((DIM, DIM), lambda i: (0, 0)),
        pl.BlockSpec((1, DIM), lambda i: (0, 0)),
        pl.BlockSpec(memory_space=pltpu.MemorySpace.HBM),
    ],
    out_specs=pl.BlockSpec((DIM, BN), lambda i: (0, i)),
    out_shape=jax.ShapeDtypeStruct((DIM, 2 * BATCH), jnp.float32),
    input_output_aliases={4: 0},
    scratch_shapes=[
        pltpu.VMEM((DIM, DIM), jnp.float32),
        pltpu.VMEM((DIM, 1), jnp.float32),
    ],
)


def kernel(emb_src, emb_tgt, W_enc, b_enc, W_dec, src_id, tgt_id):
    src_t, full_t = _sc_sweep_fn()(
        jnp.transpose(emb_src), jnp.transpose(emb_tgt),
        src_id.astype(jnp.int32), tgt_id.astype(jnp.int32))
    out_t = _mm_t(src_t, W_enc, W_dec, b_enc.reshape(1, DIM), full_t)
    return jnp.transpose(out_t)
